# baseline (speedup 1.0000x reference)
import functools

import jax
import jax.numpy as jnp
from jax import lax
from jax.experimental import pallas as pl
from jax.experimental.pallas import tpu as pltpu
from jax.experimental.pallas import tpu_sc as plsc

N = 10000
D = 128
NC = 2
NS = 16
CHUNK = 128
N_TAB = 10112
ROWS_PER_TILE = N_TAB // NS
TRASH = N


def _sc_aggregate(h, src3, dst3):
  cpt = src3.shape[1]
  mesh = plsc.VectorSubcoreMesh(core_axis_name="c", subcore_axis_name="s")

  @functools.partial(
      pl.kernel,
      out_type=jax.ShapeDtypeStruct((NC, N_TAB, D), jnp.float32),
      mesh=mesh,
      scratch_types=[
          pltpu.VMEM((cpt, CHUNK), jnp.int32),
          pltpu.VMEM((cpt, CHUNK), jnp.int32),
          pltpu.VMEM((CHUNK, D), jnp.float32),
          pltpu.VMEM_SHARED((N_TAB, D), jnp.float32),
          pltpu.SemaphoreType.DMA,
      ],
  )
  def k(h_hbm, src_hbm, dst_hbm, out_hbm, src_v, dst_v, rows_a,
        agg, sem_a):
    c = lax.axis_index("c")
    s = lax.axis_index("s")
    row = s * NC + c
    pltpu.async_copy(src_hbm.at[row], src_v, sem_a)
    pltpu.async_copy(dst_hbm.at[row], dst_v, sem_a)
    z16 = jnp.zeros((16,), jnp.float32)

    @pl.loop(0, CHUNK)
    def _(i):
      for kk in range(D // 16):
        rows_a[i, pl.ds(kk * 16, 16)] = z16

    pltpu.make_async_copy(src_hbm.at[row], src_v, sem_a).wait()
    pltpu.make_async_copy(dst_hbm.at[row], dst_v, sem_a).wait()

    base = s * ROWS_PER_TILE
    for t in range(ROWS_PER_TILE // CHUNK):
      pltpu.sync_copy(rows_a, agg.at[pl.ds(base + t * CHUNK, CHUNK)])
    rem = ROWS_PER_TILE % CHUNK
    if rem:
      pltpu.sync_copy(
          rows_a.at[pl.ds(0, rem)],
          agg.at[pl.ds(base + (ROWS_PER_TILE // CHUNK) * CHUNK, rem)])
    plsc.subcore_barrier()

    @pl.loop(0, cpt)
    def _(j):
      pltpu.async_copy(h_hbm.at[src_v.at[j]], rows_a, sem_a).wait()
      pltpu.sync_copy(rows_a, agg.at[dst_v.at[j]], add=True)

    plsc.subcore_barrier()
    pltpu.sync_copy(agg.at[pl.ds(s * ROWS_PER_TILE, ROWS_PER_TILE)],
                    out_hbm.at[c, pl.ds(s * ROWS_PER_TILE, ROWS_PER_TILE)])

  return k(h, src3, dst3)


def _mlp_body(h_ref, p0_ref, p1_ref, w1_ref, b1_ref, w2_ref, b2_ref, out_ref):
  u = (h_ref[...] + p0_ref[...].reshape(_ROW_BLK, D)
       + p1_ref[...].reshape(_ROW_BLK, D))
  t = jnp.dot(u, w1_ref[...], preferred_element_type=jnp.float32) + b1_ref[...]
  t = jax.nn.gelu(t)
  v = jnp.dot(t, w2_ref[...], preferred_element_type=jnp.float32) + b2_ref[...]
  out_ref[...] = jax.nn.gelu(v)


def _mlp_head_body(h_ref, p0_ref, p1_ref, w1_ref, b1_ref, w2_ref, b2_ref,
                   wp_ref, bp_ref, out_ref):
  u = (h_ref[...] + p0_ref[...].reshape(_ROW_BLK, D)
       + p1_ref[...].reshape(_ROW_BLK, D))
  t = jnp.dot(u, w1_ref[...], preferred_element_type=jnp.float32) + b1_ref[...]
  t = jax.nn.gelu(t)
  v = jnp.dot(t, w2_ref[...], preferred_element_type=jnp.float32) + b2_ref[...]
  g = jax.nn.gelu(v)
  out_ref[...] = (
      jnp.dot(g, wp_ref[...], preferred_element_type=jnp.float32) + bp_ref[...])


_ROW_BLK = 10000


def _row_spec():
  return pl.BlockSpec((_ROW_BLK, D), lambda i: (i, 0))


def _plane_spec(plane):
  return pl.BlockSpec((1, _ROW_BLK, D), lambda i, p=plane: (p, i, 0))


def _full_spec(shape):
  return pl.BlockSpec(shape, lambda i: tuple(0 for _ in shape))


def _tc_mlp(h, agg2, w1, b1, w2, b2):
  grid = (N // _ROW_BLK,)
  return pl.pallas_call(
      _mlp_body,
      grid=grid,
      in_specs=[_row_spec(), _plane_spec(0), _plane_spec(1),
                _full_spec((D, D)), _full_spec((1, D)),
                _full_spec((D, D)), _full_spec((1, D))],
      out_specs=_row_spec(),
      out_shape=jax.ShapeDtypeStruct((N, D), jnp.float32),
  )(h, agg2, agg2, w1, b1.reshape(1, D), w2, b2.reshape(1, D))


def _tc_mlp_head(h, agg2, w1, b1, w2, b2, wp, bp):
  grid = (N // _ROW_BLK,)
  return pl.pallas_call(
      _mlp_head_body,
      grid=grid,
      in_specs=[_row_spec(), _plane_spec(0), _plane_spec(1),
                _full_spec((D, D)), _full_spec((1, D)),
                _full_spec((D, D)), _full_spec((1, D)),
                _full_spec((D, D)), _full_spec((1, D))],
      out_specs=_row_spec(),
      out_shape=jax.ShapeDtypeStruct((N, D), jnp.float32),
  )(h, agg2, agg2, w1, b1.reshape(1, D), w2, b2.reshape(1, D), wp,
    bp.reshape(1, D))


def kernel(x, edge_index, W1a, b1a, W2a, b2a, W1b, b1b, W2b, b2b, Wp, bp):
  src = edge_index[0]
  dst = edge_index[1]
  e = src.shape[0]
  cpt = -(-e // (NC * NS * CHUNK))
  e_pad = NC * NS * cpt * CHUNK
  src_p = jnp.concatenate([src, jnp.zeros((e_pad - e,), jnp.int32)])
  pad_dst = TRASH + jnp.arange(e_pad - e, dtype=jnp.int32) % (N_TAB - N)
  dst_p = jnp.concatenate([dst, pad_dst])

  src3 = src_p.reshape(NC * NS, cpt, CHUNK)
  dst3 = dst_p.reshape(NC * NS, cpt, CHUNK)

  agg_a = _sc_aggregate(x, src3, dst3)
  h1 = _tc_mlp(x, agg_a, W1a, b1a, W2a, b2a)
  agg_b = _sc_aggregate(h1, src3, dst3)
  return _tc_mlp_head(h1, agg_b, W1b, b1b, W2b, b2b, Wp, bp)


# speedup vs baseline: 1.0038x; 1.0038x over previous
import functools

import jax
import jax.numpy as jnp
from jax import lax
from jax.experimental import pallas as pl
from jax.experimental.pallas import tpu as pltpu
from jax.experimental.pallas import tpu_sc as plsc

N = 10000
D = 128
NC = 2
NS = 16
CHUNK = 128
N_TAB = 10112
ROWS_PER_TILE = N_TAB // NS
TRASH = N


def _sc_aggregate(h, src3, dst3):
  cpt = src3.shape[1]
  mesh = plsc.VectorSubcoreMesh(core_axis_name="c", subcore_axis_name="s")

  @functools.partial(
      pl.kernel,
      out_type=jax.ShapeDtypeStruct((NC, N_TAB, D), jnp.float32),
      mesh=mesh,
      scratch_types=[
          pltpu.VMEM((cpt, CHUNK), jnp.int32),
          pltpu.VMEM((cpt, CHUNK), jnp.int32),
          pltpu.VMEM((CHUNK, D), jnp.float32),
          pltpu.VMEM_SHARED((N_TAB, D), jnp.float32),
          pltpu.SemaphoreType.DMA,
      ],
  )
  def k(h_hbm, src_hbm, dst_hbm, out_hbm, src_v, dst_v, rows_a,
        agg, sem_a):
    c = lax.axis_index("c")
    s = lax.axis_index("s")
    row = s * NC + c
    pltpu.async_copy(src_hbm.at[row], src_v, sem_a)
    pltpu.async_copy(dst_hbm.at[row], dst_v, sem_a)
    z16 = jnp.zeros((16,), jnp.float32)

    @pl.loop(0, CHUNK)
    def _(i):
      for kk in range(D // 16):
        rows_a[i, pl.ds(kk * 16, 16)] = z16

    pltpu.make_async_copy(src_hbm.at[row], src_v, sem_a).wait()
    pltpu.make_async_copy(dst_hbm.at[row], dst_v, sem_a).wait()

    base = s * ROWS_PER_TILE
    for t in range(ROWS_PER_TILE // CHUNK):
      pltpu.sync_copy(rows_a, agg.at[pl.ds(base + t * CHUNK, CHUNK)])
    rem = ROWS_PER_TILE % CHUNK
    if rem:
      pltpu.sync_copy(
          rows_a.at[pl.ds(0, rem)],
          agg.at[pl.ds(base + (ROWS_PER_TILE // CHUNK) * CHUNK, rem)])
    plsc.subcore_barrier()

    @pl.loop(0, cpt)
    def _(j):
      pltpu.async_copy(h_hbm.at[src_v.at[j]], rows_a, sem_a).wait()
      pltpu.sync_copy(rows_a, agg.at[dst_v.at[j]], add=True)

    plsc.subcore_barrier()
    pltpu.sync_copy(agg.at[pl.ds(s * ROWS_PER_TILE, ROWS_PER_TILE)],
                    out_hbm.at[c, pl.ds(s * ROWS_PER_TILE, ROWS_PER_TILE)])

  return k(h, src3, dst3)


def _mlp_body(h_ref, p0_ref, p1_ref, w1_ref, b1_ref, w2_ref, b2_ref, out_ref):
  u = (h_ref[...] + p0_ref[...].reshape(_ROW_BLK, D)
       + p1_ref[...].reshape(_ROW_BLK, D))
  t = jnp.dot(u, w1_ref[...], preferred_element_type=jnp.float32) + b1_ref[...]
  t = jax.nn.gelu(t)
  v = jnp.dot(t, w2_ref[...], preferred_element_type=jnp.float32) + b2_ref[...]
  out_ref[...] = jax.nn.gelu(v)


def _mlp_head_body(h_ref, p0_ref, p1_ref, w1_ref, b1_ref, w2_ref, b2_ref,
                   wp_ref, bp_ref, out_ref):
  u = (h_ref[...] + p0_ref[...].reshape(_ROW_BLK, D)
       + p1_ref[...].reshape(_ROW_BLK, D))
  t = jnp.dot(u, w1_ref[...], preferred_element_type=jnp.float32) + b1_ref[...]
  t = jax.nn.gelu(t)
  v = jnp.dot(t, w2_ref[...], preferred_element_type=jnp.float32) + b2_ref[...]
  g = jax.nn.gelu(v)
  out_ref[...] = (
      jnp.dot(g, wp_ref[...], preferred_element_type=jnp.float32) + bp_ref[...])


_ROW_BLK = 5000


def _row_spec():
  return pl.BlockSpec((_ROW_BLK, D), lambda i: (i, 0))


def _plane_spec(plane):
  return pl.BlockSpec((1, _ROW_BLK, D), lambda i, p=plane: (p, i, 0))


def _full_spec(shape):
  return pl.BlockSpec(shape, lambda i: tuple(0 for _ in shape))


def _tc_mlp(h, agg2, w1, b1, w2, b2):
  grid = (N // _ROW_BLK,)
  return pl.pallas_call(
      _mlp_body,
      grid=grid,
      in_specs=[_row_spec(), _plane_spec(0), _plane_spec(1),
                _full_spec((D, D)), _full_spec((1, D)),
                _full_spec((D, D)), _full_spec((1, D))],
      out_specs=_row_spec(),
      out_shape=jax.ShapeDtypeStruct((N, D), jnp.float32),
  )(h, agg2, agg2, w1, b1.reshape(1, D), w2, b2.reshape(1, D))


def _tc_mlp_head(h, agg2, w1, b1, w2, b2, wp, bp):
  grid = (N // _ROW_BLK,)
  return pl.pallas_call(
      _mlp_head_body,
      grid=grid,
      in_specs=[_row_spec(), _plane_spec(0), _plane_spec(1),
                _full_spec((D, D)), _full_spec((1, D)),
                _full_spec((D, D)), _full_spec((1, D)),
                _full_spec((D, D)), _full_spec((1, D))],
      out_specs=_row_spec(),
      out_shape=jax.ShapeDtypeStruct((N, D), jnp.float32),
  )(h, agg2, agg2, w1, b1.reshape(1, D), w2, b2.reshape(1, D), wp,
    bp.reshape(1, D))


def kernel(x, edge_index, W1a, b1a, W2a, b2a, W1b, b1b, W2b, b2b, Wp, bp):
  src = edge_index[0]
  dst = edge_index[1]
  e = src.shape[0]
  cpt = -(-e // (NC * NS * CHUNK))
  e_pad = NC * NS * cpt * CHUNK
  src_p = jnp.concatenate([src, jnp.zeros((e_pad - e,), jnp.int32)])
  pad_dst = TRASH + jnp.arange(e_pad - e, dtype=jnp.int32) % (N_TAB - N)
  dst_p = jnp.concatenate([dst, pad_dst])

  src3 = src_p.reshape(NC * NS, cpt, CHUNK)
  dst3 = dst_p.reshape(NC * NS, cpt, CHUNK)

  agg_a = _sc_aggregate(x, src3, dst3)
  h1 = _tc_mlp(x, agg_a, W1a, b1a, W2a, b2a)
  agg_b = _sc_aggregate(h1, src3, dst3)
  return _tc_mlp_head(h1, agg_b, W1b, b1b, W2b, b2b, Wp, bp)


# speedup vs baseline: 1.0069x; 1.0031x over previous
import functools

import jax
import jax.numpy as jnp
from jax import lax
from jax.experimental import pallas as pl
from jax.experimental.pallas import tpu as pltpu
from jax.experimental.pallas import tpu_sc as plsc

N = 10000
D = 128
NC = 2
NS = 16
CHUNK = 128
N_TAB = 10112
ROWS_PER_TILE = N_TAB // NS
TRASH = N


def _sc_aggregate(h, src3, dst3):
  cpt = src3.shape[1]
  mesh = plsc.VectorSubcoreMesh(core_axis_name="c", subcore_axis_name="s")

  @functools.partial(
      pl.kernel,
      out_type=jax.ShapeDtypeStruct((NC, N_TAB, D), jnp.float32),
      mesh=mesh,
      scratch_types=[
          pltpu.VMEM((cpt, CHUNK), jnp.int32),
          pltpu.VMEM((cpt, CHUNK), jnp.int32),
          pltpu.VMEM((CHUNK, D), jnp.float32),
          pltpu.VMEM_SHARED((N_TAB, D), jnp.float32),
          pltpu.SemaphoreType.DMA,
      ],
  )
  def k(h_hbm, src_hbm, dst_hbm, out_hbm, src_v, dst_v, rows_a,
        agg, sem_a):
    c = lax.axis_index("c")
    s = lax.axis_index("s")
    row = s * NC + c
    pltpu.async_copy(src_hbm.at[row], src_v, sem_a)
    pltpu.async_copy(dst_hbm.at[row], dst_v, sem_a)
    z16 = jnp.zeros((16,), jnp.float32)

    @pl.loop(0, CHUNK)
    def _(i):
      for kk in range(D // 16):
        rows_a[i, pl.ds(kk * 16, 16)] = z16

    pltpu.make_async_copy(src_hbm.at[row], src_v, sem_a).wait()
    pltpu.make_async_copy(dst_hbm.at[row], dst_v, sem_a).wait()

    base = s * ROWS_PER_TILE
    for t in range(ROWS_PER_TILE // CHUNK):
      pltpu.sync_copy(rows_a, agg.at[pl.ds(base + t * CHUNK, CHUNK)])
    rem = ROWS_PER_TILE % CHUNK
    if rem:
      pltpu.sync_copy(
          rows_a.at[pl.ds(0, rem)],
          agg.at[pl.ds(base + (ROWS_PER_TILE // CHUNK) * CHUNK, rem)])
    plsc.subcore_barrier()

    @pl.loop(0, cpt)
    def _(j):
      pltpu.async_copy(h_hbm.at[src_v.at[j]], rows_a, sem_a, priority=1).wait()
      pltpu.sync_copy(rows_a, agg.at[dst_v.at[j]], add=True)

    plsc.subcore_barrier()
    pltpu.sync_copy(agg.at[pl.ds(s * ROWS_PER_TILE, ROWS_PER_TILE)],
                    out_hbm.at[c, pl.ds(s * ROWS_PER_TILE, ROWS_PER_TILE)])

  return k(h, src3, dst3)


def _mlp_body(h_ref, p0_ref, p1_ref, w1_ref, b1_ref, w2_ref, b2_ref, out_ref):
  u = (h_ref[...] + p0_ref[...].reshape(_ROW_BLK, D)
       + p1_ref[...].reshape(_ROW_BLK, D))
  t = jnp.dot(u, w1_ref[...], preferred_element_type=jnp.float32) + b1_ref[...]
  t = jax.nn.gelu(t)
  v = jnp.dot(t, w2_ref[...], preferred_element_type=jnp.float32) + b2_ref[...]
  out_ref[...] = jax.nn.gelu(v)


def _mlp_head_body(h_ref, p0_ref, p1_ref, w1_ref, b1_ref, w2_ref, b2_ref,
                   wp_ref, bp_ref, out_ref):
  u = (h_ref[...] + p0_ref[...].reshape(_ROW_BLK, D)
       + p1_ref[...].reshape(_ROW_BLK, D))
  t = jnp.dot(u, w1_ref[...], preferred_element_type=jnp.float32) + b1_ref[...]
  t = jax.nn.gelu(t)
  v = jnp.dot(t, w2_ref[...], preferred_element_type=jnp.float32) + b2_ref[...]
  g = jax.nn.gelu(v)
  out_ref[...] = (
      jnp.dot(g, wp_ref[...], preferred_element_type=jnp.float32) + bp_ref[...])


_ROW_BLK = 5000


def _row_spec():
  return pl.BlockSpec((_ROW_BLK, D), lambda i: (i, 0))


def _plane_spec(plane):
  return pl.BlockSpec((1, _ROW_BLK, D), lambda i, p=plane: (p, i, 0))


def _full_spec(shape):
  return pl.BlockSpec(shape, lambda i: tuple(0 for _ in shape))


def _tc_mlp(h, agg2, w1, b1, w2, b2):
  grid = (N // _ROW_BLK,)
  return pl.pallas_call(
      _mlp_body,
      grid=grid,
      in_specs=[_row_spec(), _plane_spec(0), _plane_spec(1),
                _full_spec((D, D)), _full_spec((1, D)),
                _full_spec((D, D)), _full_spec((1, D))],
      out_specs=_row_spec(),
      out_shape=jax.ShapeDtypeStruct((N, D), jnp.float32),
  )(h, agg2, agg2, w1, b1.reshape(1, D), w2, b2.reshape(1, D))


def _tc_mlp_head(h, agg2, w1, b1, w2, b2, wp, bp):
  grid = (N // _ROW_BLK,)
  return pl.pallas_call(
      _mlp_head_body,
      grid=grid,
      in_specs=[_row_spec(), _plane_spec(0), _plane_spec(1),
                _full_spec((D, D)), _full_spec((1, D)),
                _full_spec((D, D)), _full_spec((1, D)),
                _full_spec((D, D)), _full_spec((1, D))],
      out_specs=_row_spec(),
      out_shape=jax.ShapeDtypeStruct((N, D), jnp.float32),
  )(h, agg2, agg2, w1, b1.reshape(1, D), w2, b2.reshape(1, D), wp,
    bp.reshape(1, D))


def kernel(x, edge_index, W1a, b1a, W2a, b2a, W1b, b1b, W2b, b2b, Wp, bp):
  src = edge_index[0]
  dst = edge_index[1]
  e = src.shape[0]
  cpt = -(-e // (NC * NS * CHUNK))
  e_pad = NC * NS * cpt * CHUNK
  src_p = jnp.concatenate([src, jnp.zeros((e_pad - e,), jnp.int32)])
  pad_dst = TRASH + jnp.arange(e_pad - e, dtype=jnp.int32) % (N_TAB - N)
  dst_p = jnp.concatenate([dst, pad_dst])

  src3 = src_p.reshape(NC * NS, cpt, CHUNK)
  dst3 = dst_p.reshape(NC * NS, cpt, CHUNK)

  agg_a = _sc_aggregate(x, src3, dst3)
  h1 = _tc_mlp(x, agg_a, W1a, b1a, W2a, b2a)
  agg_b = _sc_aggregate(h1, src3, dst3)
  return _tc_mlp_head(h1, agg_b, W1b, b1b, W2b, b2b, Wp, bp)
